# baseline (speedup 1.0000x reference)
import functools

import jax
import jax.numpy as jnp
from jax import lax
from jax.experimental import pallas as pl
from jax.experimental.pallas import tpu as pltpu
from jax.experimental.pallas import tpu_sc as plsc

B, N, D = 64, 576, 768
K = 288
BB = 8
NSPLIT = 2
BH = B // NSPLIT

NW = 32
CHUNK = 72


def _total_order_key(v):
    bits = lax.bitcast_convert_type(v, jnp.int32)
    return bits ^ ((bits >> 31) & jnp.int32(0x7FFFFFFF))


def _rank_body(noise_ref, noise_t_ref, idx_ref):
    kb = _total_order_key(noise_ref[...])
    kt = _total_order_key(noise_t_ref[0])
    ii = lax.broadcasted_iota(jnp.int32, (N, 1), 0)
    jj = lax.broadcasted_iota(jnp.int32, (1, N), 1)
    rr = lax.broadcasted_iota(jnp.int32, (1, K), 1)
    for b in range(BB):
        ki = kt[:, b:b + 1]
        kj = kb[b:b + 1, :]
        beats = (kj > ki) | ((kj == ki) & (jj < ii))
        rank = jnp.sum(beats.astype(jnp.int32), axis=1, keepdims=True)
        sel = (rank == rr).astype(jnp.int32)
        idxv = jnp.sum(sel * ii, axis=0, keepdims=True)
        row = pl.program_id(0) * BB + b
        idx_ref[b:b + 1, :] = idxv + row * N


def _topk_flat_indices(noise):
    noise_t = noise.reshape(BH // BB, BB, N).transpose(0, 2, 1)
    return pl.pallas_call(
        _rank_body,
        grid=(BH // BB,),
        in_specs=[
            pl.BlockSpec((BB, N), lambda i: (i, 0)),
            pl.BlockSpec((1, N, BB), lambda i: (i, 0, 0)),
        ],
        out_specs=pl.BlockSpec((BB, K), lambda i: (i, 0)),
        out_shape=jax.ShapeDtypeStruct((BH, K), jnp.int32),
    )(noise, noise_t)


def _sc_gather(x_flat, idx_flat):
    nrows = BH * K
    rows_per_w = nrows // NW
    nchunks = rows_per_w // CHUNK
    mesh = plsc.VectorSubcoreMesh(core_axis_name="c", subcore_axis_name="s")

    @functools.partial(
        pl.kernel,
        mesh=mesh,
        out_type=jax.ShapeDtypeStruct((nrows, D), jnp.float32),
        scratch_types=[
            pltpu.VMEM((rows_per_w,), jnp.int32),
            pltpu.VMEM((CHUNK, D), jnp.float32),
            pltpu.VMEM((CHUNK, D), jnp.float32),
            pltpu.SemaphoreType.DMA,
            pltpu.SemaphoreType.DMA,
        ],
    )
    def gather_kernel(x_hbm, idx_hbm, out_hbm, idx_v, buf0, buf1, sem0, sem1):
        wid = lax.axis_index("s") * 2 + lax.axis_index("c")
        base = wid * rows_per_w
        pltpu.sync_copy(idx_hbm.at[pl.ds(base, rows_per_w)], idx_v)
        bufs = (buf0, buf1)
        sems = (sem0, sem1)
        copies = [None] * nchunks
        copies[0] = pltpu.async_copy(
            x_hbm.at[idx_v.at[pl.ds(0, CHUNK)]], bufs[0], sems[0])
        for c in range(nchunks):
            if c + 1 < nchunks:
                copies[c + 1] = pltpu.async_copy(
                    x_hbm.at[idx_v.at[pl.ds((c + 1) * CHUNK, CHUNK)]],
                    bufs[(c + 1) % 2], sems[(c + 1) % 2])
            copies[c].wait()
            pltpu.sync_copy(bufs[c % 2],
                            out_hbm.at[pl.ds(base + c * CHUNK, CHUNK)])

    return gather_kernel(x_flat, idx_flat)


@jax.jit
def kernel(x, noise):
    outs = []
    for h in range(NSPLIT):
        nh = lax.slice_in_dim(noise, h * BH, (h + 1) * BH, axis=0)
        xh = lax.slice_in_dim(x, h * BH, (h + 1) * BH, axis=0)
        idx = _topk_flat_indices(nh)
        outs.append(_sc_gather(xh.reshape(BH * N, D), idx.reshape(BH * K)))
    return jnp.concatenate(outs, axis=0).reshape(B, K, D)


# speedup vs baseline: 1.0024x; 1.0024x over previous
import functools

import jax
import jax.numpy as jnp
from jax import lax
from jax.experimental import pallas as pl
from jax.experimental.pallas import tpu as pltpu
from jax.experimental.pallas import tpu_sc as plsc

B, N, D = 64, 576, 768
K = 288
BB = 8

NW = 32
ROWS_PER_W = (B * K) // NW
CHUNK = 72
NCHUNKS = ROWS_PER_W // CHUNK

_F32 = jnp.float32


def _total_order_key(v):
    bits = lax.bitcast_convert_type(v, jnp.int32)
    return bits ^ ((bits >> 31) & jnp.int32(0x7FFFFFFF))


def _rank_body(noise_ref, idx_ref):
    nb = noise_ref[...]
    ii = lax.broadcasted_iota(jnp.int32, (N, 1), 0)
    jj = lax.broadcasted_iota(jnp.int32, (1, N), 1)
    eye = (ii == jj).astype(_F32)
    kb = _total_order_key(nb)
    hi = lax.shift_right_logical(kb, 16).astype(_F32)
    lo = (kb & jnp.int32(0xFFFF)).astype(_F32)
    hit = lax.dot_general(eye, hi, (((1,), (1,)), ((), ())),
                          preferred_element_type=_F32)
    lot = lax.dot_general(eye, lo, (((1,), (1,)), ((), ())),
                          preferred_element_type=_F32)
    kt = (hit.astype(jnp.int32) << 16) | lot.astype(jnp.int32)
    jlt = (jj < ii).astype(_F32)
    ones_col = jnp.ones((N, 1), _F32)
    iif = ii.astype(_F32).reshape(1, N)
    rr = lax.broadcasted_iota(jnp.int32, (1, K), 1).astype(_F32)
    for b in range(BB):
        ki = kt[:, b:b + 1]
        kj = kb[b:b + 1, :]
        beats = jnp.where(kj == ki, jlt, (kj > ki).astype(_F32))
        rank = lax.dot_general(beats, ones_col, (((1,), (0,)), ((), ())),
                               preferred_element_type=_F32)
        sel = (rank == rr).astype(_F32)
        idxv = lax.dot_general(iif, sel, (((1,), (0,)), ((), ())),
                               preferred_element_type=_F32)
        row = pl.program_id(0) * BB + b
        idx_ref[b:b + 1, :] = idxv.astype(jnp.int32) + row * N


def _topk_flat_indices(noise):
    return pl.pallas_call(
        _rank_body,
        grid=(B // BB,),
        in_specs=[pl.BlockSpec((BB, N), lambda i: (i, 0))],
        out_specs=pl.BlockSpec((BB, K), lambda i: (i, 0)),
        out_shape=jax.ShapeDtypeStruct((B, K), jnp.int32),
    )(noise)


def _sc_gather(x_flat, idx_flat):
    mesh = plsc.VectorSubcoreMesh(core_axis_name="c", subcore_axis_name="s")

    @functools.partial(
        pl.kernel,
        mesh=mesh,
        out_type=jax.ShapeDtypeStruct((B * K, D), jnp.float32),
        scratch_types=[
            pltpu.VMEM((ROWS_PER_W,), jnp.int32),
            pltpu.VMEM((CHUNK, D), jnp.float32),
            pltpu.VMEM((CHUNK, D), jnp.float32),
            pltpu.SemaphoreType.DMA,
            pltpu.SemaphoreType.DMA,
        ],
    )
    def gather_kernel(x_hbm, idx_hbm, out_hbm, idx_v, buf0, buf1, sem0, sem1):
        wid = lax.axis_index("s") * 2 + lax.axis_index("c")
        base = wid * ROWS_PER_W
        pltpu.sync_copy(idx_hbm.at[pl.ds(base, ROWS_PER_W)], idx_v)
        bufs = (buf0, buf1)
        sems = (sem0, sem1)
        copies = [None] * NCHUNKS
        copies[0] = pltpu.async_copy(
            x_hbm.at[idx_v.at[pl.ds(0, CHUNK)]], bufs[0], sems[0])
        for c in range(NCHUNKS):
            if c + 1 < NCHUNKS:
                copies[c + 1] = pltpu.async_copy(
                    x_hbm.at[idx_v.at[pl.ds((c + 1) * CHUNK, CHUNK)]],
                    bufs[(c + 1) % 2], sems[(c + 1) % 2])
            copies[c].wait()
            pltpu.sync_copy(bufs[c % 2],
                            out_hbm.at[pl.ds(base + c * CHUNK, CHUNK)])

    return gather_kernel(x_flat, idx_flat)


@jax.jit
def kernel(x, noise):
    idx = _topk_flat_indices(noise)
    out_flat = _sc_gather(x.reshape(B * N, D), idx.reshape(B * K))
    return out_flat.reshape(B, K, D)


# speedup vs baseline: 2.0026x; 1.9978x over previous
import functools

import jax
import jax.numpy as jnp
from jax import lax
from jax.experimental import pallas as pl
from jax.experimental.pallas import tpu as pltpu
from jax.experimental.pallas import tpu_sc as plsc

B, N, D = 64, 576, 768
K = 288
BB = 8

NW = 32
ROWS_PER_W = (B * K) // NW
CHUNK = 64
NCHUNKS = ROWS_PER_W // CHUNK

_F32 = jnp.float32


def _total_order_key(v):
    bits = lax.bitcast_convert_type(v, jnp.int32)
    return bits ^ ((bits >> 31) & jnp.int32(0x7FFFFFFF))


def _rank_body(noise_ref, noise_t_ref, idx_ref):
    kb = _total_order_key(noise_ref[...])
    kt = _total_order_key(noise_t_ref[0])
    ii = lax.broadcasted_iota(jnp.int32, (N, 1), 0)
    jj = lax.broadcasted_iota(jnp.int32, (1, N), 1)
    jlt = (jj < ii).astype(jnp.int32)
    rr = lax.broadcasted_iota(jnp.int32, (1, K), 1)
    for b in range(BB):
        ki = kt[:, b:b + 1]
        kj = kb[b:b + 1, :]
        beats = jnp.where(kj == ki, jlt, (kj > ki).astype(jnp.int32))
        rank = jnp.sum(beats, axis=1, keepdims=True)
        sel = (rank == rr).astype(jnp.int32)
        idxv = jnp.sum(sel * ii, axis=0, keepdims=True)
        row = pl.program_id(0) * BB + b
        idx_ref[b:b + 1, :] = idxv + row * N


def _topk_flat_indices(noise):
    noise_t = noise.reshape(B // BB, BB, N).transpose(0, 2, 1)
    return pl.pallas_call(
        _rank_body,
        grid=(B // BB,),
        in_specs=[
            pl.BlockSpec((BB, N), lambda i: (i, 0)),
            pl.BlockSpec((1, N, BB), lambda i: (i, 0, 0)),
        ],
        out_specs=pl.BlockSpec((BB, K), lambda i: (i, 0)),
        out_shape=jax.ShapeDtypeStruct((B, K), jnp.int32),
    )(noise, noise_t)


def _sc_gather(x_flat, idx_flat):
    mesh = plsc.VectorSubcoreMesh(core_axis_name="c", subcore_axis_name="s")

    @functools.partial(
        pl.kernel,
        mesh=mesh,
        out_type=jax.ShapeDtypeStruct((B * K, D), jnp.float32),
        scratch_types=[
            pltpu.VMEM((ROWS_PER_W,), jnp.int32),
            pltpu.VMEM((CHUNK, D), jnp.float32),
            pltpu.VMEM((CHUNK, D), jnp.float32),
            pltpu.SemaphoreType.DMA,
            pltpu.SemaphoreType.DMA,
        ],
    )
    def gather_kernel(x_hbm, idx_hbm, out_hbm, idx_v, buf0, buf1, sem0, sem1):
        wid = lax.axis_index("s") * 2 + lax.axis_index("c")
        base = wid * ROWS_PER_W
        pltpu.sync_copy(idx_hbm.at[pl.ds(base, ROWS_PER_W)], idx_v)
        bufs = (buf0, buf1)
        sems = (sem0, sem1)
        copies = [None] * NCHUNKS
        copies[0] = pltpu.async_copy(
            x_hbm.at[idx_v.at[pl.ds(0, CHUNK)]], bufs[0], sems[0])
        for c in range(NCHUNKS):
            if c + 1 < NCHUNKS:
                copies[c + 1] = pltpu.async_copy(
                    x_hbm.at[idx_v.at[pl.ds((c + 1) * CHUNK, CHUNK)]],
                    bufs[(c + 1) % 2], sems[(c + 1) % 2])
            copies[c].wait()
            pltpu.sync_copy(bufs[c % 2],
                            out_hbm.at[pl.ds(base + c * CHUNK, CHUNK)])

    return gather_kernel(x_flat, idx_flat)


@jax.jit
def kernel(x, noise):
    idx = _topk_flat_indices(noise)
    out_flat = _sc_gather(x.reshape(B * N, D), idx.reshape(B * K))
    return out_flat.reshape(B, K, D)


# speedup vs baseline: 2.2869x; 1.1420x over previous
import functools

import jax
import jax.numpy as jnp
from jax import lax
from jax.experimental import pallas as pl
from jax.experimental.pallas import tpu as pltpu
from jax.experimental.pallas import tpu_sc as plsc

B, N, D = 64, 576, 768
K = 288
BB = 8
BH = B // 2

NW = 32
ROWS_PER_W = (BH * K) // NW
CHUNKS = (64, 64, 64, 64, 32)


def _total_order_key(v):
    bits = lax.bitcast_convert_type(v, jnp.int32)
    return bits ^ ((bits >> 31) & jnp.int32(0x7FFFFFFF))


def _make_rank_body(row_off):
    def _rank_body(noise_ref, noise_t_ref, idx_ref):
        kb = _total_order_key(noise_ref[...])
        kt = _total_order_key(noise_t_ref[0])
        ii = lax.broadcasted_iota(jnp.int32, (N, 1), 0)
        jj = lax.broadcasted_iota(jnp.int32, (1, N), 1)
        rr = lax.broadcasted_iota(jnp.int32, (1, K), 1)
        for b in range(BB):
            ki = kt[:, b:b + 1]
            kj = kb[b:b + 1, :]
            beats = (kj > ki) | ((kj == ki) & (jj < ii))
            rank = jnp.sum(beats.astype(jnp.int32), axis=1, keepdims=True)
            sel = (rank == rr).astype(jnp.int32)
            idxv = jnp.sum(sel * ii, axis=0, keepdims=True)
            row = row_off + pl.program_id(0) * BB + b
            idx_ref[b:b + 1, :] = idxv + row * N
    return _rank_body


def _topk_flat_indices(noise_h, row_off):
    noise_t = noise_h.reshape(BH // BB, BB, N).transpose(0, 2, 1)
    return pl.pallas_call(
        _make_rank_body(row_off),
        grid=(BH // BB,),
        in_specs=[
            pl.BlockSpec((BB, N), lambda i: (i, 0)),
            pl.BlockSpec((1, N, BB), lambda i: (i, 0, 0)),
        ],
        out_specs=pl.BlockSpec((BB, K), lambda i: (i, 0)),
        out_shape=jax.ShapeDtypeStruct((BH, K), jnp.int32),
    )(noise_h, noise_t)


_SCRATCH = [
    pltpu.VMEM((ROWS_PER_W,), jnp.int32),
    pltpu.VMEM((CHUNKS[0], D), jnp.float32),
    pltpu.VMEM((CHUNKS[0], D), jnp.float32),
    pltpu.SemaphoreType.DMA,
    pltpu.SemaphoreType.DMA,
]


def _gather_worker(x_hbm, idx_hbm, out_hbm, idx_v, buf0, buf1, sem0, sem1,
                   out_off):
    wid = lax.axis_index("s") * 2 + lax.axis_index("c")
    base = wid * ROWS_PER_W
    pltpu.sync_copy(idx_hbm.at[pl.ds(base, ROWS_PER_W)], idx_v)
    bufs = (buf0, buf1)
    sems = (sem0, sem1)
    offs = [0]
    for c in CHUNKS:
        offs.append(offs[-1] + c)
    nch = len(CHUNKS)
    copies = [None] * nch
    copies[0] = pltpu.async_copy(
        x_hbm.at[idx_v.at[pl.ds(0, CHUNKS[0])]],
        bufs[0].at[pl.ds(0, CHUNKS[0])], sems[0])
    for c in range(nch):
        if c + 1 < nch:
            copies[c + 1] = pltpu.async_copy(
                x_hbm.at[idx_v.at[pl.ds(offs[c + 1], CHUNKS[c + 1])]],
                bufs[(c + 1) % 2].at[pl.ds(0, CHUNKS[c + 1])],
                sems[(c + 1) % 2])
        copies[c].wait()
        pltpu.sync_copy(
            bufs[c % 2].at[pl.ds(0, CHUNKS[c])],
            out_hbm.at[pl.ds(out_off + base + offs[c], CHUNKS[c])])


def _sc_gather_first(x_flat, idx_flat):
    mesh = plsc.VectorSubcoreMesh(core_axis_name="c", subcore_axis_name="s")

    @functools.partial(
        pl.kernel, mesh=mesh,
        out_type=jax.ShapeDtypeStruct((B * K, D), jnp.float32),
        scratch_types=_SCRATCH,
    )
    def gather_kernel(x_hbm, idx_hbm, out_hbm, *scr):
        _gather_worker(x_hbm, idx_hbm, out_hbm, *scr, out_off=0)

    return gather_kernel(x_flat, idx_flat)


def _sc_gather_second(x_flat, idx_flat, out_ref):
    mesh = plsc.VectorSubcoreMesh(core_axis_name="c", subcore_axis_name="s")

    @functools.partial(
        pl.kernel, mesh=mesh,
        scratch_types=_SCRATCH,
    )
    def gather_kernel(x_hbm, idx_hbm, out_hbm, *scr):
        _gather_worker(x_hbm, idx_hbm, out_hbm, *scr, out_off=BH * K)

    gather_kernel(x_flat, idx_flat, out_ref)


@jax.jit
def kernel(x, noise):
    x_flat = x.reshape(B * N, D)
    idx1 = _topk_flat_indices(noise[:BH], 0)
    idx2 = _topk_flat_indices(noise[BH:], BH)
    out = _sc_gather_first(x_flat, idx1.reshape(BH * K))
    out_ref = jax.new_ref(out)
    _sc_gather_second(x_flat, idx2.reshape(BH * K), out_ref)
    return out_ref[...].reshape(B, K, D)
